# Initial kernel scaffold; baseline (speedup 1.0000x reference)
import jax
import jax.numpy as jnp
from jax.experimental import pallas as pl

B, D, E = 32768, 768, 64
BLK = 2048


def _route_kernel(x_ref, w_ref, weights_ref, idx_ref):
    w = w_ref[...]
    inv_norm = jax.lax.rsqrt(jnp.maximum(jnp.sum(w * w, axis=1), 1e-24))
    sim = jax.lax.dot_general(
        x_ref[...], w,
        dimension_numbers=(((1,), (1,)), ((), ())),
        preferred_element_type=jnp.float32,
    )
    sim = sim * inv_norm[None, :]
    idx = jnp.argmax(sim, axis=1).astype(jnp.int32)
    eids = jax.lax.broadcasted_iota(jnp.int32, sim.shape, 1)
    weights_ref[...] = (eids == idx[:, None]).astype(jnp.float32)
    idx_ref[...] = idx[:, None]


@jax.jit
def kernel(language_token, routing_embeddings):
    weights, idx = pl.pallas_call(
        _route_kernel,
        grid=(B // BLK,),
        in_specs=[
            pl.BlockSpec((BLK, D), lambda i: (i, 0)),
            pl.BlockSpec((E, D), lambda i: (0, 0)),
        ],
        out_specs=[
            pl.BlockSpec((BLK, E), lambda i: (i, 0)),
            pl.BlockSpec((BLK, 1), lambda i: (i, 0)),
        ],
        out_shape=[
            jax.ShapeDtypeStruct((B, E), jnp.float32),
            jax.ShapeDtypeStruct((B, 1), jnp.int32),
        ],
    )(language_token, routing_embeddings)
    return (weights, idx)


# speedup vs baseline: 2.9104x; 2.9104x over previous
import jax
import jax.numpy as jnp
from jax.experimental import pallas as pl

B, D, E = 32768, 768, 64
BLK = 2048


def _l2n(v):
    n = jnp.sqrt(jnp.sum(v * v, axis=1, keepdims=True))
    return v / jnp.maximum(n, 1e-12)


def _route_kernel(x_ref, w_ref, weights_ref, idx_ref):
    nx = _l2n(x_ref[...]).astype(jnp.bfloat16)
    nw = _l2n(w_ref[...]).astype(jnp.bfloat16)
    sim = jax.lax.dot_general(
        nx, nw,
        dimension_numbers=(((1,), (1,)), ((), ())),
        preferred_element_type=jnp.float32,
    )
    idx = jnp.argmax(sim, axis=1).astype(jnp.int32)
    eids = jax.lax.broadcasted_iota(jnp.int32, sim.shape, 1)
    weights_ref[...] = (eids == idx[:, None]).astype(jnp.float32)
    idx_ref[...] = idx[:, None]


@jax.jit
def kernel(language_token, routing_embeddings):
    weights, idx = pl.pallas_call(
        _route_kernel,
        grid=(B // BLK,),
        in_specs=[
            pl.BlockSpec((BLK, D), lambda i: (i, 0)),
            pl.BlockSpec((E, D), lambda i: (0, 0)),
        ],
        out_specs=[
            pl.BlockSpec((BLK, E), lambda i: (i, 0)),
            pl.BlockSpec((BLK, 1), lambda i: (i, 0)),
        ],
        out_shape=[
            jax.ShapeDtypeStruct((B, E), jnp.float32),
            jax.ShapeDtypeStruct((B, 1), jnp.int32),
        ],
    )(language_token, routing_embeddings)
    return (weights, idx)
